# baseline (speedup 1.0000x reference)
import functools

import jax
import jax.numpy as jnp
from jax import lax
from jax.experimental import pallas as pl
from jax.experimental.pallas import tpu as pltpu
from jax.experimental.pallas import tpu_sc as plsc

SMIN = 0.01
SMAX = 50.0
NUM_LAYERS = 2
MB = 64

SC_NC = 2
SC_NS = 16
SC_NW = SC_NC * SC_NS


def _sc_gather(table, idx, n_rows, D):
    b_per_w = n_rows // SC_NW
    mesh = plsc.VectorSubcoreMesh(core_axis_name="c", subcore_axis_name="s")

    @functools.partial(
        pl.kernel, mesh=mesh,
        out_type=jax.ShapeDtypeStruct((n_rows, D), jnp.float32),
        scratch_types=[
            pltpu.VMEM((b_per_w,), jnp.int32),
            pltpu.VMEM((b_per_w, D), jnp.float32),
            pltpu.SemaphoreType.DMA,
        ],
    )
    def k(table_hbm, idx_hbm, out_hbm, idx_v, rows_v, sem):
        wid = lax.axis_index("s") * SC_NC + lax.axis_index("c")
        base = wid * b_per_w
        pltpu.sync_copy(idx_hbm.at[pl.ds(base, b_per_w)], idx_v)
        pltpu.async_copy(table_hbm.at[idx_v], rows_v, sem).wait()
        pltpu.sync_copy(rows_v, out_hbm.at[pl.ds(base, b_per_w)])

    return k(table, idx)


def _roll_block(v, o, mb, n):
    if o == 0:
        return v
    d = v.shape[-1]
    v3 = v.reshape(mb, n, d)
    return jnp.concatenate([v3[:, o:, :], v3[:, :o, :]], axis=1).reshape(mb * n, d)


def _step(t_ref, h0_ref, pos_ref, noise_ref, Wt_ref, A_ref, B_ref,
          C_ref, Wc_ref, Wn_ref, bn_ref, out_ref, *, mb, n, D, n_total):
    NB = mb * n

    t_nodes = t_ref[...]
    std = SMIN * (SMAX / SMIN) ** t_nodes

    noise = noise_ref[...]
    noise3 = noise.reshape(mb, n, 3)
    noise_c = (noise3 - jnp.mean(noise3, axis=1, keepdims=True)).reshape(NB, 3)

    x = pos_ref[...] + noise_c * std

    h = h0_ref[...] + t_nodes * Wt_ref[0][None, :]

    score = jnp.zeros((NB, 3), dtype=jnp.float32)
    for l in range(NUM_LAYERS):
        a = h * A_ref[l][None, :]
        b = h * B_ref[l][None, :]
        Cl = C_ref[l][None, :]
        Wcl = Wc_ref[l].reshape(D, 1)
        agg_m = jnp.zeros((NB, D), dtype=jnp.float32)
        agg_x = jnp.zeros((NB, 3), dtype=jnp.float32)
        for o in range(n):
            a_rot = _roll_block(a, o, mb, n)
            x_rot = _roll_block(x, o, mb, n)
            rel = x_rot - x
            d2 = jnp.sum(rel * rel, axis=1, keepdims=True)
            m = jax.nn.silu(a_rot + b + d2 * Cl)
            agg_m = agg_m + m
            coef = jax.lax.dot_general(m, Wcl, (((1,), (0,)), ((), ())),
                                       preferred_element_type=jnp.float32)
            agg_x = agg_x + rel * coef
        agg_x = agg_x / n
        h = h + jax.nn.silu(
            jax.lax.dot_general(agg_m, Wn_ref[l], (((1,), (0,)), ((), ())),
                                preferred_element_type=jnp.float32)
            + bn_ref[l][None, :])
        x = x + agg_x
        score = score + agg_x

    score = score / std
    score3 = score.reshape(mb, n, 3)
    score = (score3 - jnp.mean(score3, axis=1, keepdims=True)).reshape(NB, 3)
    r = score * std + noise_c
    partial = jnp.sum(r * r, axis=(0, 1), keepdims=True) / n_total

    @pl.when(pl.program_id(0) == 0)
    def _init():
        out_ref[...] = jnp.zeros((1, 1), jnp.float32)

    out_ref[...] += partial


def kernel(pos, atomic_numbers, mask, atom_emb, W_t, A, Bv, C, Wc, Wn, bn):
    B = mask.shape[0]
    N = pos.shape[0]
    n = N // B
    D = atom_emb.shape[1]

    kt = jax.random.fold_in(jax.random.key(0), 1)
    kn = jax.random.fold_in(jax.random.key(0), 2)
    t = jax.random.uniform(kt, (B,), minval=1e-3, maxval=1.0, dtype=jnp.float32)
    noise = jax.random.normal(kn, (N, 3), dtype=jnp.float32)

    t_nodes = jnp.repeat(t, n).reshape(N, 1)
    Wt2 = W_t.reshape(1, D)

    h0 = _sc_gather(atom_emb, atomic_numbers, N, D)

    mb = MB
    grid = B // mb
    NB = mb * n
    full = lambda g: (0, 0)
    out = pl.pallas_call(
        functools.partial(_step, mb=mb, n=n, D=D, n_total=N),
        grid=(grid,),
        in_specs=[
            pl.BlockSpec((NB, 1), lambda g: (g, 0)),
            pl.BlockSpec((NB, D), lambda g: (g, 0)),
            pl.BlockSpec((NB, 3), lambda g: (g, 0)),
            pl.BlockSpec((NB, 3), lambda g: (g, 0)),
            pl.BlockSpec((1, D), full),
            pl.BlockSpec((NUM_LAYERS, D), full),
            pl.BlockSpec((NUM_LAYERS, D), full),
            pl.BlockSpec((NUM_LAYERS, D), full),
            pl.BlockSpec((NUM_LAYERS, D), full),
            pl.BlockSpec((NUM_LAYERS, D, D), lambda g: (0, 0, 0)),
            pl.BlockSpec((NUM_LAYERS, D), full),
        ],
        out_specs=pl.BlockSpec((1, 1), full),
        out_shape=jax.ShapeDtypeStruct((1, 1), jnp.float32),
    )(t_nodes, h0, pos, noise, Wt2, A, Bv, C, Wc, Wn, bn)
    return out[0, 0]


# speedup vs baseline: 1.0388x; 1.0388x over previous
import functools

import jax
import jax.numpy as jnp
from jax import lax
from jax.experimental import pallas as pl
from jax.experimental.pallas import tpu as pltpu
from jax.experimental.pallas import tpu_sc as plsc

SMIN = 0.01
SMAX = 50.0
NUM_LAYERS = 2
MB = 64

SC_NC = 2
SC_NS = 16
SC_NW = SC_NC * SC_NS


def _sc_gather(table, idx, n_rows, D):
    b_per_w = n_rows // SC_NW
    mesh = plsc.VectorSubcoreMesh(core_axis_name="c", subcore_axis_name="s")

    @functools.partial(
        pl.kernel, mesh=mesh,
        out_type=jax.ShapeDtypeStruct((n_rows, D), jnp.float32),
        scratch_types=[
            pltpu.VMEM((b_per_w,), jnp.int32),
            pltpu.VMEM((b_per_w, D), jnp.float32),
            pltpu.SemaphoreType.DMA,
        ],
    )
    def k(table_hbm, idx_hbm, out_hbm, idx_v, rows_v, sem):
        wid = lax.axis_index("s") * SC_NC + lax.axis_index("c")
        base = wid * b_per_w
        pltpu.sync_copy(idx_hbm.at[pl.ds(base, b_per_w)], idx_v)
        pltpu.async_copy(table_hbm.at[idx_v], rows_v, sem).wait()
        pltpu.sync_copy(rows_v, out_hbm.at[pl.ds(base, b_per_w)])

    return k(table, idx)


def _roll_block(v, o, mb, n):
    if o == 0:
        return v
    d = v.shape[-1]
    v3 = v.reshape(mb, n, d)
    return jnp.concatenate([v3[:, o:, :], v3[:, :o, :]], axis=1).reshape(mb * n, d)


def _step(t_ref, h0_ref, pos_ref, noise_ref, Wt_ref, A_ref, B_ref,
          C_ref, Wc_ref, Wn_ref, bn_ref, out_ref, *, mb, n, D, n_total):
    NB = mb * n

    t_nodes = t_ref[...]
    std = SMIN * (SMAX / SMIN) ** t_nodes

    noise = noise_ref[...]
    noise3 = noise.reshape(mb, n, 3)
    noise_c = (noise3 - jnp.mean(noise3, axis=1, keepdims=True)).reshape(NB, 3)

    x = pos_ref[...] + noise_c * std

    h = h0_ref[...] + t_nodes * Wt_ref[0][None, :]

    score = jnp.zeros((NB, 3), dtype=jnp.float32)
    for l in range(NUM_LAYERS):
        a = h * A_ref[l][None, :]
        b = h * B_ref[l][None, :]
        Cl = C_ref[l][None, :]
        Wcl = Wc_ref[l].reshape(D, 1)
        a3 = a.reshape(mb, n, D)
        b3 = b.reshape(mb, n, D)
        x3 = x.reshape(mb, n, 3)
        agg_m = jnp.zeros((NB, D), dtype=jnp.float32)
        agg_x = jnp.zeros((NB, 3), dtype=jnp.float32)
        for i in range(n):
            a_i = jnp.broadcast_to(a3[:, i:i + 1, :], (mb, n, D))
            x_i = jnp.broadcast_to(x3[:, i:i + 1, :], (mb, n, 3))
            rel = (x_i - x3).reshape(NB, 3)
            d2 = jnp.sum(rel * rel, axis=1, keepdims=True)
            m = jax.nn.silu(a_i.reshape(NB, D) + b + d2 * Cl)
            agg_m = agg_m + m
            coef = jax.lax.dot_general(m, Wcl, (((1,), (0,)), ((), ())),
                                       preferred_element_type=jnp.float32)
            agg_x = agg_x + rel * coef
        agg_x = agg_x / n
        h = h + jax.nn.silu(
            jax.lax.dot_general(agg_m, Wn_ref[l], (((1,), (0,)), ((), ())),
                                preferred_element_type=jnp.float32)
            + bn_ref[l][None, :])
        x = x + agg_x
        score = score + agg_x

    score = score / std
    score3 = score.reshape(mb, n, 3)
    score = (score3 - jnp.mean(score3, axis=1, keepdims=True)).reshape(NB, 3)
    r = score * std + noise_c
    partial = jnp.sum(r * r, axis=(0, 1), keepdims=True) / n_total

    @pl.when(pl.program_id(0) == 0)
    def _init():
        out_ref[...] = jnp.zeros((1, 1), jnp.float32)

    out_ref[...] += partial


def kernel(pos, atomic_numbers, mask, atom_emb, W_t, A, Bv, C, Wc, Wn, bn):
    B = mask.shape[0]
    N = pos.shape[0]
    n = N // B
    D = atom_emb.shape[1]

    kt = jax.random.fold_in(jax.random.key(0), 1)
    kn = jax.random.fold_in(jax.random.key(0), 2)
    t = jax.random.uniform(kt, (B,), minval=1e-3, maxval=1.0, dtype=jnp.float32)
    noise = jax.random.normal(kn, (N, 3), dtype=jnp.float32)

    t_nodes = jnp.repeat(t, n).reshape(N, 1)
    Wt2 = W_t.reshape(1, D)

    h0 = _sc_gather(atom_emb, atomic_numbers, N, D)

    mb = MB
    grid = B // mb
    NB = mb * n
    full = lambda g: (0, 0)
    out = pl.pallas_call(
        functools.partial(_step, mb=mb, n=n, D=D, n_total=N),
        grid=(grid,),
        in_specs=[
            pl.BlockSpec((NB, 1), lambda g: (g, 0)),
            pl.BlockSpec((NB, D), lambda g: (g, 0)),
            pl.BlockSpec((NB, 3), lambda g: (g, 0)),
            pl.BlockSpec((NB, 3), lambda g: (g, 0)),
            pl.BlockSpec((1, D), full),
            pl.BlockSpec((NUM_LAYERS, D), full),
            pl.BlockSpec((NUM_LAYERS, D), full),
            pl.BlockSpec((NUM_LAYERS, D), full),
            pl.BlockSpec((NUM_LAYERS, D), full),
            pl.BlockSpec((NUM_LAYERS, D, D), lambda g: (0, 0, 0)),
            pl.BlockSpec((NUM_LAYERS, D), full),
        ],
        out_specs=pl.BlockSpec((1, 1), full),
        out_shape=jax.ShapeDtypeStruct((1, 1), jnp.float32),
    )(t_nodes, h0, pos, noise, Wt2, A, Bv, C, Wc, Wn, bn)
    return out[0, 0]
